# baseline (speedup 1.0000x reference)
import dataclasses

import jax
import jax.numpy as jnp
from jax.experimental import pallas as pl
from jax.experimental.pallas import tpu as pltpu
from jax.experimental.pallas import tpu_sc as plsc

N = 256
MARGIN = 1.0
BIG = 1e30
LANES = 16
NCHUNK = N // LANES
NTILES = 32
DN = 64
SCN = N - DN
APT = SCN // NTILES // 2
ROWS = 2 * APT
DCHUNK = 16


def _prep_body(x_ref, y_ref, lab_ref,
               axp_ref, ayp_ref, bx_ref, by_ref, cnt_ref):
    x = x_ref[:]
    y = y_ref[:]
    lab = lab_ref[:]

    g = jnp.dot(x, y.T, preferred_element_type=jnp.float32)
    nx = jnp.sum(x * x, axis=1, keepdims=True)
    ny = jnp.sum(y * y, axis=1, keepdims=True)
    d = nx + ny.T - 2.0 * g
    dt = d.T

    same = lab.T == lab
    ii = jax.lax.broadcasted_iota(jnp.int32, (N, N), 0)
    jj = jax.lax.broadcasted_iota(jnp.int32, (N, N), 1)
    pm = same & (ii < jj)

    axp_ref[...] = jnp.where(pm, d + MARGIN, -BIG).reshape(N * N)
    ayp_ref[...] = jnp.where(pm, dt + MARGIN, -BIG).reshape(N * N)
    bx_ref[...] = jnp.where(same, BIG, d).reshape(N * N)
    by_ref[...] = jnp.where(same, BIG, dt).reshape(N * N)

    npos = jnp.sum(pm.astype(jnp.float32), axis=1)
    nneg = jnp.sum((~same).astype(jnp.float32), axis=1)
    cnt_ref[:, :] = jnp.broadcast_to(jnp.sum(npos * nneg), (1, 1))


def _dense_body(x_ref, y_ref, lab_ref, out_ref):
    x = x_ref[:]
    y = y_ref[:]
    lab = lab_ref[:]
    labd = lab[:, :DN]

    nx = jnp.sum(x * x, axis=1, keepdims=True)
    ny = jnp.sum(y * y, axis=1, keepdims=True)
    gx = jnp.dot(x[:DN, :], y.T, preferred_element_type=jnp.float32)
    gy = jnp.dot(y[:DN, :], x.T, preferred_element_type=jnp.float32)
    d = nx[:DN, :] + ny.T - 2.0 * gx
    dt = ny[:DN, :] + nx.T - 2.0 * gy

    same = labd.T == lab
    ii = jax.lax.broadcasted_iota(jnp.int32, (DN, N), 0)
    jj = jax.lax.broadcasted_iota(jnp.int32, (DN, N), 1)
    pm = same & (ii < jj)

    axp = jnp.where(pm, d + MARGIN, -BIG)
    ayp = jnp.where(pm, dt + MARGIN, -BIG)
    bx = jnp.where(same, BIG, d)
    by = jnp.where(same, BIG, dt)

    total = jnp.float32(0.0)
    for c in range(0, DN, DCHUNK):
        sx = jnp.sum(jax.nn.relu(axp[c:c + DCHUNK, :, None]
                                 - bx[c:c + DCHUNK, None, :]))
        sy = jnp.sum(jax.nn.relu(ayp[c:c + DCHUNK, :, None]
                                 - by[c:c + DCHUNK, None, :]))
        total = total + sx + sy
    out_ref[:, :] = jnp.broadcast_to(total, (1, 1))


def _sc_loss_body(axp_hbm, ayp_hbm, bx_hbm, by_hbm, out_hbm,
                  axp_vm, ayp_vm, bx_vm, by_vm, alx_vm, aly_vm, stage_vm, sem):
    core = jax.lax.axis_index("c")
    sub = jax.lax.axis_index("s")
    t = core * 16 + sub
    lane = jax.lax.iota(jnp.int32, LANES)

    front0 = DN + t * APT
    back0 = N - APT - t * APT

    copies = []
    for hbm, vm in ((axp_hbm, axp_vm), (ayp_hbm, ayp_vm),
                    (bx_hbm, bx_vm), (by_hbm, by_vm)):
        copies.append(pltpu.async_copy(
            hbm.at[pl.ds(front0 * N, APT * N)], vm.at[pl.ds(0, APT * N)], sem))
        copies.append(pltpu.async_copy(
            hbm.at[pl.ds(back0 * N, APT * N)],
            vm.at[pl.ds(APT * N, APT * N)], sem))
    for cp in copies:
        cp.wait()

    accx = jnp.zeros((LANES,), jnp.float32)
    accy = jnp.zeros((LANES,), jnp.float32)

    NWAY = 4
    GRP = NCHUNK // NWAY
    REG = 64
    for row in range(ROWS):
        offs = [jnp.zeros((LANES,), jnp.int32) for _ in range(NWAY)]
        for c in range(NCHUNK):
            w = c // GRP
            sl = pl.ds(row * N + c * LANES, LANES)
            axc = axp_vm[sl]
            ayc = ayp_vm[sl]
            m = axc > 0.0
            mi = jnp.where(m, jnp.int32(1), jnp.int32(0))
            tgt = offs[w] + plsc.cumsum(mi) + (w * REG - 1)
            plsc.store_scatter(alx_vm, [tgt], axc, mask=m)
            plsc.store_scatter(aly_vm, [tgt], ayc, mask=m)
            offs[w] = offs[w] + plsc.all_reduce_population_count(m)
        cnts = [jnp.max(o) for o in offs]

        bxv = [bx_vm[pl.ds(row * N + c * LANES, LANES)] for c in range(NCHUNK)]
        byv = [by_vm[pl.ds(row * N + c * LANES, LANES)] for c in range(NCHUNK)]

        def pair_body(p, car, bxv=bxv, byv=byv, cnts=cnts):
            ax1, ay1 = car
            idx = p
            base = jnp.int32(0)
            for w in range(1, NWAY):
                past = p >= sum(cnts[:w])
                idx = jnp.where(past, p - sum(cnts[:w]), idx)
                base = jnp.where(past, jnp.int32(w * REG), base)
            psp = jnp.full((LANES,), base + idx, jnp.int32)
            a_x = plsc.load_gather(alx_vm, [psp])
            a_y = plsc.load_gather(aly_vm, [psp])
            sx = jnp.maximum(a_x - bxv[0], 0.0)
            sy = jnp.maximum(a_y - byv[0], 0.0)
            for c in range(1, NCHUNK):
                sx = sx + jnp.maximum(a_x - bxv[c], 0.0)
                sy = sy + jnp.maximum(a_y - byv[c], 0.0)
            return ax1 + sx, ay1 + sy

        accx, accy = jax.lax.fori_loop(
            0, sum(cnts), pair_body, (accx, accy))

    total = jnp.sum(accx) + jnp.sum(accy)
    stage_vm[:] = jnp.where(lane == 0, total, 0.0)
    pltpu.async_copy(stage_vm, out_hbm.at[t], sem).wait()


def kernel(embeddings_x, embeddings_y, labels):
    lab2d = labels.reshape(1, N)
    scmat = jax.ShapeDtypeStruct((N * N,), jnp.float32)
    axp, ayp, bx, by, cnt = pl.pallas_call(
        _prep_body,
        out_shape=[scmat, scmat, scmat, scmat,
                   jax.ShapeDtypeStruct((1, 1), jnp.float32)],
    )(embeddings_x, embeddings_y, lab2d)

    cp = pltpu.CompilerParams()
    if "needs_layout_passes" in pltpu.CompilerParams.__dataclass_fields__:
        cp = dataclasses.replace(cp, needs_layout_passes=False)
    mesh = plsc.VectorSubcoreMesh(core_axis_name="c", subcore_axis_name="s")
    sc_loss = pl.kernel(
        _sc_loss_body,
        out_type=jax.ShapeDtypeStruct((NTILES, LANES), jnp.float32),
        mesh=mesh,
        compiler_params=cp,
        scratch_types=[
            pltpu.VMEM((ROWS * N,), jnp.float32),
            pltpu.VMEM((ROWS * N,), jnp.float32),
            pltpu.VMEM((ROWS * N,), jnp.float32),
            pltpu.VMEM((ROWS * N,), jnp.float32),
            pltpu.VMEM((N + LANES,), jnp.float32),
            pltpu.VMEM((N + LANES,), jnp.float32),
            pltpu.VMEM((LANES,), jnp.float32),
            pltpu.SemaphoreType.DMA,
        ],
    )
    partials = sc_loss(axp, ayp, bx, by)

    dense_part = pl.pallas_call(
        _dense_body,
        out_shape=jax.ShapeDtypeStruct((1, 1), jnp.float32),
    )(embeddings_x, embeddings_y, lab2d)

    return (jnp.sum(partials[:, 0]) + dense_part[0, 0]) / cnt[0, 0]


# speedup vs baseline: 1.1020x; 1.1020x over previous
import dataclasses

import jax
import jax.numpy as jnp
from jax.experimental import pallas as pl
from jax.experimental.pallas import tpu as pltpu
from jax.experimental.pallas import tpu_sc as plsc

N = 256
MARGIN = 1.0
BIG = 1e30
LANES = 16
NCHUNK = N // LANES
NTILES = 32
DN = 64
SCN = N - DN
APT = SCN // NTILES // 2
ROWS = 2 * APT
DCHUNK = 16


def _prep_body(x_ref, y_ref, lab_ref,
               axp_ref, ayp_ref, bx_ref, by_ref, cnt_ref):
    x = x_ref[:]
    y = y_ref[:]
    lab = lab_ref[:]

    g = jnp.dot(x, y.T, preferred_element_type=jnp.float32)
    nx = jnp.sum(x * x, axis=1, keepdims=True)
    ny = jnp.sum(y * y, axis=1, keepdims=True)
    d = nx + ny.T - 2.0 * g
    dt = d.T

    same = lab.T == lab
    ii = jax.lax.broadcasted_iota(jnp.int32, (N, N), 0)
    jj = jax.lax.broadcasted_iota(jnp.int32, (N, N), 1)
    pm = same & (ii < jj)

    axp_ref[...] = jnp.where(pm, d + MARGIN, -BIG).reshape(N * N)
    ayp_ref[...] = jnp.where(pm, dt + MARGIN, -BIG).reshape(N * N)
    bx_ref[...] = jnp.where(same, BIG, d).reshape(N * N)
    by_ref[...] = jnp.where(same, BIG, dt).reshape(N * N)

    npos = jnp.sum(pm.astype(jnp.float32), axis=1)
    nneg = jnp.sum((~same).astype(jnp.float32), axis=1)
    cnt_ref[:, :] = jnp.broadcast_to(jnp.sum(npos * nneg), (1, 1))


def _dense_body(x_ref, y_ref, lab_ref, out_ref):
    x = x_ref[:]
    y = y_ref[:]
    lab = lab_ref[:]
    labd = lab[:, :DN]

    nx = jnp.sum(x * x, axis=1, keepdims=True)
    ny = jnp.sum(y * y, axis=1, keepdims=True)
    gx = jnp.dot(x[:DN, :], y.T, preferred_element_type=jnp.float32)
    gy = jnp.dot(y[:DN, :], x.T, preferred_element_type=jnp.float32)
    d = nx[:DN, :] + ny.T - 2.0 * gx
    dt = ny[:DN, :] + nx.T - 2.0 * gy

    same = labd.T == lab
    ii = jax.lax.broadcasted_iota(jnp.int32, (DN, N), 0)
    jj = jax.lax.broadcasted_iota(jnp.int32, (DN, N), 1)
    pm = same & (ii < jj)

    axp = jnp.where(pm, d + MARGIN, -BIG)
    ayp = jnp.where(pm, dt + MARGIN, -BIG)
    bx = jnp.where(same, BIG, d)
    by = jnp.where(same, BIG, dt)

    total = jnp.float32(0.0)
    for c in range(0, DN, DCHUNK):
        sx = jnp.sum(jax.nn.relu(axp[c:c + DCHUNK, :, None]
                                 - bx[c:c + DCHUNK, None, :]))
        sy = jnp.sum(jax.nn.relu(ayp[c:c + DCHUNK, :, None]
                                 - by[c:c + DCHUNK, None, :]))
        total = total + sx + sy
    out_ref[:, :] = jnp.broadcast_to(total, (1, 1))


def _sc_loss_body(axp_hbm, ayp_hbm, bx_hbm, by_hbm, out_hbm,
                  axp_vm, ayp_vm, bx_vm, by_vm, alx_vm, aly_vm, stage_vm, sem):
    core = jax.lax.axis_index("c")
    sub = jax.lax.axis_index("s")
    t = core * 16 + sub
    lane = jax.lax.iota(jnp.int32, LANES)

    front0 = DN + t * APT
    back0 = N - APT - t * APT

    copies = []
    for hbm, vm in ((axp_hbm, axp_vm), (ayp_hbm, ayp_vm),
                    (bx_hbm, bx_vm), (by_hbm, by_vm)):
        copies.append(pltpu.async_copy(
            hbm.at[pl.ds(front0 * N, APT * N)], vm.at[pl.ds(0, APT * N)], sem))
        copies.append(pltpu.async_copy(
            hbm.at[pl.ds(back0 * N, APT * N)],
            vm.at[pl.ds(APT * N, APT * N)], sem))
    for cp in copies:
        cp.wait()

    accx = jnp.zeros((LANES,), jnp.float32)
    accy = jnp.zeros((LANES,), jnp.float32)

    HALF = NCHUNK // 2
    for row in range(ROWS):
        offva = jnp.zeros((LANES,), jnp.int32)
        offvb = jnp.zeros((LANES,), jnp.int32)
        for c in range(NCHUNK):
            sl = pl.ds(row * N + c * LANES, LANES)
            axc = axp_vm[sl]
            ayc = ayp_vm[sl]
            m = axc > 0.0
            mi = jnp.where(m, jnp.int32(1), jnp.int32(0))
            if c < HALF:
                tgt = offva + plsc.cumsum(mi) - 1
            else:
                tgt = offvb + plsc.cumsum(mi) + (128 - 1)
            plsc.store_scatter(alx_vm, [tgt], axc, mask=m)
            plsc.store_scatter(aly_vm, [tgt], ayc, mask=m)
            if c < HALF:
                offva = offva + plsc.all_reduce_population_count(m)
            else:
                offvb = offvb + plsc.all_reduce_population_count(m)
        offa = jnp.max(offva)
        offb = jnp.max(offvb)

        bxv = [bx_vm[pl.ds(row * N + c * LANES, LANES)] for c in range(NCHUNK)]
        byv = [by_vm[pl.ds(row * N + c * LANES, LANES)] for c in range(NCHUNK)]

        def pair_body(p, car, bxv=bxv, byv=byv, offa=offa):
            ax1, ay1 = car
            idx = jnp.where(p < offa, p, p - offa + 128)
            psp = jnp.full((LANES,), idx, jnp.int32)
            a_x = plsc.load_gather(alx_vm, [psp])
            a_y = plsc.load_gather(aly_vm, [psp])
            sx = jnp.maximum(a_x - bxv[0], 0.0)
            sy = jnp.maximum(a_y - byv[0], 0.0)
            for c in range(1, NCHUNK):
                sx = sx + jnp.maximum(a_x - bxv[c], 0.0)
                sy = sy + jnp.maximum(a_y - byv[c], 0.0)
            return ax1 + sx, ay1 + sy

        accx, accy = jax.lax.fori_loop(0, offa + offb, pair_body, (accx, accy))

    total = jnp.sum(accx) + jnp.sum(accy)
    stage_vm[:] = jnp.where(lane == 0, total, 0.0)
    pltpu.async_copy(stage_vm, out_hbm.at[t], sem).wait()


def kernel(embeddings_x, embeddings_y, labels):
    lab2d = labels.reshape(1, N)
    scmat = jax.ShapeDtypeStruct((N * N,), jnp.float32)
    axp, ayp, bx, by, cnt = pl.pallas_call(
        _prep_body,
        out_shape=[scmat, scmat, scmat, scmat,
                   jax.ShapeDtypeStruct((1, 1), jnp.float32)],
    )(embeddings_x, embeddings_y, lab2d)

    cp = pltpu.CompilerParams()
    if "needs_layout_passes" in pltpu.CompilerParams.__dataclass_fields__:
        cp = dataclasses.replace(cp, needs_layout_passes=False)
    mesh = plsc.VectorSubcoreMesh(core_axis_name="c", subcore_axis_name="s")
    sc_loss = pl.kernel(
        _sc_loss_body,
        out_type=jax.ShapeDtypeStruct((NTILES, LANES), jnp.float32),
        mesh=mesh,
        compiler_params=cp,
        scratch_types=[
            pltpu.VMEM((ROWS * N,), jnp.float32),
            pltpu.VMEM((ROWS * N,), jnp.float32),
            pltpu.VMEM((ROWS * N,), jnp.float32),
            pltpu.VMEM((ROWS * N,), jnp.float32),
            pltpu.VMEM((N + LANES,), jnp.float32),
            pltpu.VMEM((N + LANES,), jnp.float32),
            pltpu.VMEM((LANES,), jnp.float32),
            pltpu.SemaphoreType.DMA,
        ],
    )
    partials = sc_loss(axp, ayp, bx, by)

    dense_part = pl.pallas_call(
        _dense_body,
        out_shape=jax.ShapeDtypeStruct((1, 1), jnp.float32),
    )(embeddings_x, embeddings_y, lab2d)

    return (jnp.sum(partials[:, 0]) + dense_part[0, 0]) / cnt[0, 0]
